# baseline (device time: 21414 ns/iter reference)
import jax
import jax.numpy as jnp
from jax import lax
from jax.experimental import pallas as pl
from jax.experimental.pallas import tpu as pltpu


def kernel(x, assign, W1, W2):
    t, d = x.shape
    n_exp, _, f = W1.shape
    qt = t // 4
    q_out = 2 * lax.axis_index("x") + lax.axis_index("y")
    xq = lax.dynamic_slice_in_dim(x, q_out * qt, qt).astype(jnp.bfloat16)
    w1b = W1.astype(jnp.bfloat16)
    w2b = W2.astype(jnp.bfloat16)

    def body(x_ref, a_ref, w1_ref, w2_ref, out_ref,
             xr_ref, ar_ref, ys_ref, yr_ref, s_ref, sf_ref, sems):
        my_x = lax.axis_index("x")
        my_y = lax.axis_index("y")
        my_z = lax.axis_index("z")
        zpeer = (my_x, my_y, 1 - my_z)
        ynbr = (my_x, 1 - my_y, my_z)
        xnbr = (1 - my_x, my_y, my_z)
        diag = (1 - my_x, 1 - my_y, my_z)
        q = 2 * my_x + my_y
        qsl = pl.ds(q * qt, qt)
        aq1d = pl.ds(q * qt, qt)

        barrier = pltpu.get_barrier_semaphore()
        for nbr in (zpeer, ynbr, xnbr, diag):
            pl.semaphore_signal(barrier, inc=1, device_id=nbr,
                                device_id_type=pl.DeviceIdType.MESH)
        pl.semaphore_wait(barrier, 4)

        rz1x = pltpu.make_async_remote_copy(
            src_ref=x_ref, dst_ref=xr_ref,
            send_sem=sems.at[0], recv_sem=sems.at[1],
            device_id=zpeer, device_id_type=pl.DeviceIdType.MESH)
        rz1x.start()
        rz1a = pltpu.make_async_remote_copy(
            src_ref=a_ref.at[aq1d], dst_ref=ar_ref,
            send_sem=sems.at[2], recv_sem=sems.at[3],
            device_id=zpeer, device_id_type=pl.DeviceIdType.MESH)
        rz1a.start()

        e_base = 2 * my_z

        def ffn(x_blk, a_blk):
            m = x_blk.shape[0]
            a_col = a_blk.reshape(m, 1)
            acc = jnp.zeros((m, d), jnp.float32)
            for el in range(n_exp):
                mask = a_col == (e_base + el)
                xm = jnp.where(mask, x_blk, jnp.bfloat16(0))
                h = jnp.maximum(
                    jnp.dot(xm, w1_ref[el],
                            preferred_element_type=jnp.float32), 0.0)
                acc = acc + jnp.dot(
                    h.astype(jnp.bfloat16), w2_ref[el],
                    preferred_element_type=jnp.float32)
            return acc

        s_ref[qsl, :] = ffn(x_ref[...], a_ref[aq1d]).astype(jnp.bfloat16)

        rz1x.wait()
        rz1a.wait()
        ys_ref[...] = ffn(xr_ref[...], ar_ref[...]).astype(jnp.bfloat16)
        rz2 = pltpu.make_async_remote_copy(
            src_ref=ys_ref, dst_ref=yr_ref,
            send_sem=sems.at[4], recv_sem=sems.at[5],
            device_id=zpeer, device_id_type=pl.DeviceIdType.MESH)
        rz2.start()
        rz2.wait()

        s_ref[qsl, :] = s_ref[qsl, :] + yr_ref[...]

        swaps = []
        for k, nbr in enumerate((ynbr, xnbr, diag)):
            r = pltpu.make_async_remote_copy(
                src_ref=s_ref.at[qsl, :], dst_ref=s_ref.at[qsl, :],
                send_sem=sems.at[6 + 2 * k], recv_sem=sems.at[7 + 2 * k],
                device_id=nbr, device_id_type=pl.DeviceIdType.MESH)
            r.start()
            swaps.append(r)
        for r in swaps:
            r.wait()

        sf_ref[...] = s_ref[...].astype(jnp.float32)
        st = pltpu.make_async_copy(sf_ref, out_ref, sems.at[12])
        st.start()
        st.wait()

    return pl.pallas_call(
        body,
        out_shape=jax.ShapeDtypeStruct((t, d), jnp.float32),
        in_specs=[
            pl.BlockSpec(memory_space=pltpu.VMEM),
            pl.BlockSpec(memory_space=pltpu.VMEM),
            pl.BlockSpec(memory_space=pltpu.VMEM),
            pl.BlockSpec(memory_space=pltpu.VMEM),
        ],
        out_specs=pl.BlockSpec(memory_space=pltpu.MemorySpace.HBM),
        scratch_shapes=[
            pltpu.VMEM((qt, d), jnp.bfloat16),
            pltpu.VMEM((qt,), jnp.int32),
            pltpu.VMEM((qt, d), jnp.bfloat16),
            pltpu.VMEM((qt, d), jnp.bfloat16),
            pltpu.VMEM((t, d), jnp.bfloat16),
            pltpu.VMEM((t, d), jnp.float32),
            pltpu.SemaphoreType.DMA((13,)),
        ],
        compiler_params=pltpu.CompilerParams(collective_id=0),
    )(xq, assign, w1b, w2b)


# device time: 20513 ns/iter; 1.0439x vs baseline; 1.0439x over previous
import jax
import jax.numpy as jnp
from jax import lax
from jax.experimental import pallas as pl
from jax.experimental.pallas import tpu as pltpu


def kernel(x, assign, W1, W2):
    t, d = x.shape
    n_exp, _, f = W1.shape
    qt = t // 4
    q_out = 2 * lax.axis_index("x") + lax.axis_index("y")
    xq = lax.dynamic_slice_in_dim(x, q_out * qt, qt).astype(jnp.bfloat16)
    w1b = W1.astype(jnp.bfloat16)
    w2b = W2.astype(jnp.bfloat16)

    def body(x_ref, a_ref, w1_ref, w2_ref, out_ref,
             xr_ref, ar_ref, ys_ref, yr_ref, s_ref, sf_ref, sems):
        my_x = lax.axis_index("x")
        my_y = lax.axis_index("y")
        my_z = lax.axis_index("z")
        zpeer = (my_x, my_y, 1 - my_z)
        ynbr = (my_x, 1 - my_y, my_z)
        xnbr = (1 - my_x, my_y, my_z)
        diag = (1 - my_x, 1 - my_y, my_z)
        q = 2 * my_x + my_y
        qsl = pl.ds(q * qt, qt)
        aq1d = pl.ds(q * qt, qt)

        barrier = pltpu.get_barrier_semaphore()
        for nbr in (zpeer, ynbr, xnbr, diag):
            pl.semaphore_signal(barrier, inc=1, device_id=nbr,
                                device_id_type=pl.DeviceIdType.MESH)
        pl.semaphore_wait(barrier, 4)

        ht = qt // 2
        sub = lambda ref, c: ref.at[pl.ds(c * ht, ht), :]
        rz1 = []
        for c in range(2):
            r = pltpu.make_async_remote_copy(
                src_ref=sub(x_ref, c), dst_ref=sub(xr_ref, c),
                send_sem=sems.at[0 + c], recv_sem=sems.at[2 + c],
                device_id=zpeer, device_id_type=pl.DeviceIdType.MESH)
            r.start()
            rz1.append(r)
        rz1a = pltpu.make_async_remote_copy(
            src_ref=a_ref.at[aq1d], dst_ref=ar_ref,
            send_sem=sems.at[4], recv_sem=sems.at[5],
            device_id=zpeer, device_id_type=pl.DeviceIdType.MESH)
        rz1a.start()

        e_base = 2 * my_z

        def ffn(x_blk, a_blk):
            m = x_blk.shape[0]
            a_col = a_blk.reshape(m, 1)
            acc = jnp.zeros((m, d), jnp.float32)
            for el in range(n_exp):
                mask = a_col == (e_base + el)
                xm = jnp.where(mask, x_blk, jnp.bfloat16(0))
                h = jnp.maximum(
                    jnp.dot(xm, w1_ref[el],
                            preferred_element_type=jnp.float32), 0.0)
                acc = acc + jnp.dot(
                    h.astype(jnp.bfloat16), w2_ref[el],
                    preferred_element_type=jnp.float32)
            return acc

        s_ref[qsl, :] = ffn(x_ref[...], a_ref[aq1d]).astype(jnp.bfloat16)

        rz1a.wait()
        rz2 = []
        for c in range(2):
            rz1[c].wait()
            ys_ref[pl.ds(c * ht, ht), :] = ffn(
                xr_ref[pl.ds(c * ht, ht), :],
                ar_ref[pl.ds(c * ht, ht)]).astype(jnp.bfloat16)
            r = pltpu.make_async_remote_copy(
                src_ref=sub(ys_ref, c), dst_ref=sub(yr_ref, c),
                send_sem=sems.at[6 + c], recv_sem=sems.at[8 + c],
                device_id=zpeer, device_id_type=pl.DeviceIdType.MESH)
            r.start()
            rz2.append(r)
        for r in rz2:
            r.wait()

        s_ref[qsl, :] = s_ref[qsl, :] + yr_ref[...]

        swaps = []
        for k, nbr in enumerate((ynbr, xnbr, diag)):
            r = pltpu.make_async_remote_copy(
                src_ref=s_ref.at[qsl, :], dst_ref=s_ref.at[qsl, :],
                send_sem=sems.at[10 + 2 * k], recv_sem=sems.at[11 + 2 * k],
                device_id=nbr, device_id_type=pl.DeviceIdType.MESH)
            r.start()
            swaps.append(r)
        for r in swaps:
            r.wait()

        sf_ref[...] = s_ref[...].astype(jnp.float32)
        st = pltpu.make_async_copy(sf_ref, out_ref, sems.at[16])
        st.start()
        st.wait()

    return pl.pallas_call(
        body,
        out_shape=jax.ShapeDtypeStruct((t, d), jnp.float32),
        in_specs=[
            pl.BlockSpec(memory_space=pltpu.VMEM),
            pl.BlockSpec(memory_space=pltpu.VMEM),
            pl.BlockSpec(memory_space=pltpu.VMEM),
            pl.BlockSpec(memory_space=pltpu.VMEM),
        ],
        out_specs=pl.BlockSpec(memory_space=pltpu.MemorySpace.HBM),
        scratch_shapes=[
            pltpu.VMEM((qt, d), jnp.bfloat16),
            pltpu.VMEM((qt,), jnp.int32),
            pltpu.VMEM((qt, d), jnp.bfloat16),
            pltpu.VMEM((qt, d), jnp.bfloat16),
            pltpu.VMEM((t, d), jnp.bfloat16),
            pltpu.VMEM((t, d), jnp.float32),
            pltpu.SemaphoreType.DMA((17,)),
        ],
        compiler_params=pltpu.CompilerParams(collective_id=0),
    )(xq, assign, w1b, w2b)
